# baseline (speedup 1.0000x reference)
import functools

import jax
import jax.numpy as jnp
from jax import lax
from jax.experimental import pallas as pl
from jax.experimental.pallas import tpu as pltpu
from jax.experimental.pallas import tpu_sc as plsc

NUM_ITEMS = 1_000_000
TBL = 1 << 20
TR, TC_ = TBL // 128, 128
ROWS, COLS = 16384, 200
NC, NS = 2, 16
NW = NC * NS
ROWS_PER_W = ROWS // NW
RW = 64
NWIN = ROWS_PER_W // RW


def _table_body(x_ref, o_ref):
    x = x_ref[...]
    prev_rows = jnp.concatenate(
        [jnp.full((1, TC_), -1e30, jnp.float32), x[:-1, :]], axis=0
    )
    col = lax.broadcasted_iota(jnp.int32, (TR, TC_), 1)
    row = lax.broadcasted_iota(jnp.int32, (TR, TC_), 0)
    xsel = jnp.where(col == TC_ - 1, prev_rows, x)
    shifted = pltpu.roll(xsel, 1, axis=1)
    tbl = jax.nn.sigmoid(8.0 * shifted)
    o_ref[...] = jnp.where((row == 0) & (col == 0), 0.0, tbl).reshape(TBL)


N_IDX = ROWS * COLS
WIN = RW * COLS
@functools.partial(
    pl.kernel,
    out_type=jax.ShapeDtypeStruct((N_IDX,), jnp.float32),
    mesh=plsc.VectorSubcoreMesh(core_axis_name="c", subcore_axis_name="s"),
    scratch_types=[
        pltpu.VMEM_SHARED((TBL,), jnp.float32),
        pltpu.VMEM((WIN,), jnp.int32),
        pltpu.VMEM((WIN,), jnp.int32),
        pltpu.VMEM((WIN,), jnp.float32),
        pltpu.VMEM((WIN,), jnp.float32),
        pltpu.SemaphoreType.DMA,
        pltpu.SemaphoreType.DMA,
        pltpu.SemaphoreType.DMA,
        pltpu.SemaphoreType.DMA,
        pltpu.SemaphoreType.DMA,
        pltpu.SemaphoreType.DMA,
    ],
)
def _gather(table_hbm, idx_hbm, out_hbm, tbl_sp,
            idx_v0, idx_v1, out_v0, out_v1,
            in_s0, in_s1, g_s0, g_s1, st_s0, st_s1):
    c = lax.axis_index("c")
    s = lax.axis_index("s")
    wid = s * NC + c
    base = wid * (N_IDX // NW)

    idx_v = (idx_v0, idx_v1)
    out_v = (out_v0, out_v1)
    in_s = (in_s0, in_s1)
    g_s = (g_s0, g_s1)
    st_s = (st_s0, st_s1)

    in_h = [None, None]
    st_h = [None, None]
    in_h[0] = pltpu.async_copy(idx_hbm.at[pl.ds(base, WIN)], idx_v[0], in_s[0])

    seg = TBL // NS
    pltpu.sync_copy(table_hbm.at[pl.ds(s * seg, seg)], tbl_sp.at[pl.ds(s * seg, seg)])
    plsc.subcore_barrier()

    for w in range(NWIN):
        b = w & 1
        if w + 1 < NWIN:
            off = base + (w + 1) * WIN
            in_h[1 - b] = pltpu.async_copy(
                idx_hbm.at[pl.ds(off, WIN)], idx_v[1 - b], in_s[1 - b]
            )
        in_h[b].wait()
        if w >= 2:
            st_h[b].wait()
        pltpu.async_copy(tbl_sp.at[idx_v[b]], out_v[b], g_s[b]).wait()
        st_h[b] = pltpu.async_copy(
            out_v[b], out_hbm.at[pl.ds(base + w * WIN, WIN)], st_s[b]
        )
    st_h[0].wait()
    st_h[1].wait()


def kernel(inputs, item_rating_logits):
    pad = jnp.full((TBL - NUM_ITEMS + 1,), -1e30, jnp.float32)
    x = jnp.concatenate([item_rating_logits, pad]).reshape(TR, TC_)
    table = pl.pallas_call(
        _table_body,
        out_shape=jax.ShapeDtypeStruct((TBL,), jnp.float32),
    )(x)
    idx = jnp.bitwise_and(inputs.reshape(N_IDX), TBL - 1)
    out = _gather(table, idx)
    return jnp.abs(out.reshape(ROWS, COLS))


# speedup vs baseline: 1.1200x; 1.1200x over previous
import functools

import jax
import jax.numpy as jnp
from jax import lax
from jax.experimental import pallas as pl
from jax.experimental.pallas import tpu as pltpu
from jax.experimental.pallas import tpu_sc as plsc

NUM_ITEMS = 1_000_000
TBL = 1 << 20
TR, TC_ = TBL // 128, 128
ROWS, COLS = 16384, 200
NC, NS = 2, 16
NW = NC * NS
ROWS_PER_W = ROWS // NW
RW = 32
NWIN = ROWS_PER_W // RW


def _table_body(x_ref, o_ref):
    x = x_ref[...]
    prev_rows = jnp.concatenate(
        [jnp.full((1, TC_), -1e30, jnp.float32), x[:-1, :]], axis=0
    )
    col = lax.broadcasted_iota(jnp.int32, (TR, TC_), 1)
    row = lax.broadcasted_iota(jnp.int32, (TR, TC_), 0)
    xsel = jnp.where(col == TC_ - 1, prev_rows, x)
    shifted = pltpu.roll(xsel, 1, axis=1)
    tbl = jax.nn.sigmoid(8.0 * shifted)
    o_ref[...] = jnp.where((row == 0) & (col == 0), 0.0, tbl).reshape(TBL)


N_IDX = ROWS * COLS
WIN = RW * COLS
@functools.partial(
    pl.kernel,
    out_type=jax.ShapeDtypeStruct((N_IDX,), jnp.float32),
    mesh=plsc.VectorSubcoreMesh(core_axis_name="c", subcore_axis_name="s"),
    scratch_types=[
        pltpu.VMEM_SHARED((TBL,), jnp.float32),
        pltpu.VMEM((WIN,), jnp.int32),
        pltpu.VMEM((WIN,), jnp.int32),
        pltpu.VMEM((WIN,), jnp.float32),
        pltpu.VMEM((WIN,), jnp.float32),
        pltpu.SemaphoreType.DMA,
        pltpu.SemaphoreType.DMA,
        pltpu.SemaphoreType.DMA,
        pltpu.SemaphoreType.DMA,
        pltpu.SemaphoreType.DMA,
        pltpu.SemaphoreType.DMA,
    ],
)
def _gather(table_hbm, idx_hbm, out_hbm, tbl_sp,
            idx_v0, idx_v1, out_v0, out_v1,
            in_s0, in_s1, g_s0, g_s1, st_s0, st_s1):
    c = lax.axis_index("c")
    s = lax.axis_index("s")
    wid = s * NC + c
    base = wid * (N_IDX // NW)

    idx_v = (idx_v0, idx_v1)
    out_v = (out_v0, out_v1)
    in_s = (in_s0, in_s1)
    g_s = (g_s0, g_s1)
    st_s = (st_s0, st_s1)

    in_h = [None, None]
    st_h = [None, None]
    in_h[0] = pltpu.async_copy(idx_hbm.at[pl.ds(base, WIN)], idx_v[0], in_s[0])

    seg = TBL // NS
    pltpu.sync_copy(table_hbm.at[pl.ds(s * seg, seg)], tbl_sp.at[pl.ds(s * seg, seg)])
    plsc.subcore_barrier()

    for w in range(NWIN):
        b = w & 1
        if w + 1 < NWIN:
            off = base + (w + 1) * WIN
            in_h[1 - b] = pltpu.async_copy(
                idx_hbm.at[pl.ds(off, WIN)], idx_v[1 - b], in_s[1 - b]
            )
        in_h[b].wait()
        if w >= 2:
            st_h[b].wait()
        pltpu.async_copy(tbl_sp.at[idx_v[b]], out_v[b], g_s[b]).wait()
        st_h[b] = pltpu.async_copy(
            out_v[b], out_hbm.at[pl.ds(base + w * WIN, WIN)], st_s[b]
        )
    st_h[0].wait()
    st_h[1].wait()


def kernel(inputs, item_rating_logits):
    pad = jnp.full((TBL - NUM_ITEMS + 1,), -1e30, jnp.float32)
    x = jnp.concatenate([item_rating_logits, pad]).reshape(TR, TC_)
    table = pl.pallas_call(
        _table_body,
        out_shape=jax.ShapeDtypeStruct((TBL,), jnp.float32),
    )(x)
    out = _gather(table, inputs.reshape(N_IDX))
    return out.reshape(ROWS, COLS)


# speedup vs baseline: 1.1320x; 1.0107x over previous
import functools

import jax
import jax.numpy as jnp
from jax import lax
from jax.experimental import pallas as pl
from jax.experimental.pallas import tpu as pltpu
from jax.experimental.pallas import tpu_sc as plsc

NUM_ITEMS = 1_000_000
TBL = 1 << 20
TR, TC_ = TBL // 128, 128
ROWS, COLS = 16384, 200
NC, NS = 2, 16
NW = NC * NS
ROWS_PER_W = ROWS // NW
RW = 64
NWIN = ROWS_PER_W // RW


def _table_body(x_ref, o_ref):
    x = x_ref[...]
    prev_rows = jnp.concatenate(
        [jnp.full((1, TC_), -1e30, jnp.float32), x[:-1, :]], axis=0
    )
    col = lax.broadcasted_iota(jnp.int32, (TR, TC_), 1)
    row = lax.broadcasted_iota(jnp.int32, (TR, TC_), 0)
    xsel = jnp.where(col == TC_ - 1, prev_rows, x)
    shifted = pltpu.roll(xsel, 1, axis=1)
    tbl = jax.nn.sigmoid(8.0 * shifted)
    o_ref[...] = jnp.where((row == 0) & (col == 0), 0.0, tbl).reshape(TBL)


N_IDX = ROWS * COLS
WIN = RW * COLS
@functools.partial(
    pl.kernel,
    out_type=jax.ShapeDtypeStruct((N_IDX,), jnp.float32),
    mesh=plsc.VectorSubcoreMesh(core_axis_name="c", subcore_axis_name="s"),
    scratch_types=[
        pltpu.VMEM_SHARED((TBL,), jnp.float32),
        pltpu.VMEM((WIN,), jnp.int32),
        pltpu.VMEM((WIN,), jnp.int32),
        pltpu.VMEM((WIN,), jnp.float32),
        pltpu.VMEM((WIN,), jnp.float32),
        pltpu.SemaphoreType.DMA,
        pltpu.SemaphoreType.DMA,
        pltpu.SemaphoreType.DMA,
        pltpu.SemaphoreType.DMA,
        pltpu.SemaphoreType.DMA,
        pltpu.SemaphoreType.DMA,
    ],
)
def _gather(table_hbm, idx_hbm, out_hbm, tbl_sp,
            idx_v0, idx_v1, out_v0, out_v1,
            in_s0, in_s1, g_s0, g_s1, st_s0, st_s1):
    c = lax.axis_index("c")
    s = lax.axis_index("s")
    wid = s * NC + c
    base = wid * (N_IDX // NW)

    idx_v = (idx_v0, idx_v1)
    out_v = (out_v0, out_v1)
    in_s = (in_s0, in_s1)
    g_s = (g_s0, g_s1)
    st_s = (st_s0, st_s1)

    in_h = [None, None]
    st_h = [None, None]
    in_h[0] = pltpu.async_copy(idx_hbm.at[pl.ds(base, WIN)], idx_v[0], in_s[0])

    seg = TBL // NS
    pltpu.sync_copy(table_hbm.at[pl.ds(s * seg, seg)], tbl_sp.at[pl.ds(s * seg, seg)])
    plsc.subcore_barrier()

    for w in range(NWIN):
        b = w & 1
        if w + 1 < NWIN:
            off = base + (w + 1) * WIN
            in_h[1 - b] = pltpu.async_copy(
                idx_hbm.at[pl.ds(off, WIN)], idx_v[1 - b], in_s[1 - b]
            )
        in_h[b].wait()
        if w >= 2:
            st_h[b].wait()
        pltpu.async_copy(tbl_sp.at[idx_v[b]], out_v[b], g_s[b]).wait()
        st_h[b] = pltpu.async_copy(
            out_v[b], out_hbm.at[pl.ds(base + w * WIN, WIN)], st_s[b]
        )
    st_h[0].wait()
    st_h[1].wait()


def kernel(inputs, item_rating_logits):
    pad = jnp.full((TBL - NUM_ITEMS + 1,), -1e30, jnp.float32)
    x = jnp.concatenate([item_rating_logits, pad]).reshape(TR, TC_)
    table = pl.pallas_call(
        _table_body,
        out_shape=jax.ShapeDtypeStruct((TBL,), jnp.float32),
    )(x)
    out = _gather(table, inputs.reshape(N_IDX))
    return out.reshape(ROWS, COLS)
